# baseline (speedup 1.0000x reference)
import functools

import jax
import jax.numpy as jnp
from jax import lax
from jax.experimental import pallas as pl
from jax.experimental.pallas import tpu as pltpu
from jax.experimental.pallas import tpu_sc as plsc

B = 16384
D_NODE = 256
D_MEM = 512
D_EMB = 256

NC = 2
NS = 16
NW = NC * NS
CH = 32
SPLITS = (10240, 6144)
NBUF = 4
OFFSETS = tuple(sum(SPLITS[:k]) for k in range(len(SPLITS)))
assert sum(SPLITS) == B

_mesh = plsc.VectorSubcoreMesh(core_axis_name="c", subcore_axis_name="s")


def _make_sc_gather(offset, bs):
    bpw = bs // NW
    nchunk = bpw // CH

    def body(nf_hbm, mem_hbm, idx_hbm, gnf_hbm, gmem_hbm,
             idx_v, nf_buf, mem_buf, *sems):
        wid = lax.axis_index("s") * NC + lax.axis_index("c")
        base = wid * bpw
        pltpu.sync_copy(idx_hbm.at[pl.ds(offset + base, bpw)], idx_v)
        sem_g = sems[:NBUF]
        sem_w = sems[NBUF:]

        def fire_gather(c, p):
            ix = idx_v.at[pl.ds(c * CH, CH)]
            return (pltpu.async_copy(nf_hbm.at[ix], nf_buf.at[p], sem_g[p]),
                    pltpu.async_copy(mem_hbm.at[ix], mem_buf.at[p], sem_g[p]))

        def fire_write(c, p):
            o = base + c * CH
            return (pltpu.async_copy(nf_buf.at[p], gnf_hbm.at[pl.ds(o, CH)], sem_w[p]),
                    pltpu.async_copy(mem_buf.at[p], gmem_hbm.at[pl.ds(o, CH)], sem_w[p]))

        gather_cps = [None] * NBUF
        write_cps = [None] * NBUF
        for p in range(min(NBUF, nchunk)):
            gather_cps[p] = fire_gather(p, p)
        for c in range(nchunk):
            p = c % NBUF
            for cp in gather_cps[p]:
                cp.wait()
            write_cps[p] = fire_write(c, p)
            nxt = c + NBUF
            if nxt < nchunk:
                for cp in write_cps[p]:
                    cp.wait()
                gather_cps[p] = fire_gather(nxt, p)
        for p in range(NBUF):
            if write_cps[p] is not None:
                for cp in write_cps[p]:
                    cp.wait()

    return functools.partial(
        pl.kernel,
        out_type=(
            jax.ShapeDtypeStruct((bs, D_NODE), jnp.float32),
            jax.ShapeDtypeStruct((bs, D_MEM), jnp.float32),
        ),
        mesh=_mesh,
        scratch_types=[
            pltpu.VMEM((bpw,), jnp.int32),
            pltpu.VMEM((NBUF, CH, D_NODE), jnp.float32),
            pltpu.VMEM((NBUF, CH, D_MEM), jnp.float32),
        ] + [pltpu.SemaphoreType.DMA] * (2 * NBUF),
    )(body)


_sc_gathers = [_make_sc_gather(OFFSETS[k], SPLITS[k]) for k in range(len(SPLITS))]


TB = 2048


def _mm_body(gnf_ref, gmem_ref, wn_ref, wm_ref, o_ref):
    o_ref[...] = (
        jnp.dot(gnf_ref[...], wn_ref[...], preferred_element_type=jnp.float32)
        + jnp.dot(gmem_ref[...], wm_ref[...], preferred_element_type=jnp.float32)
    )


def _mm_body_aliased(gnf_ref, gmem_ref, wn_ref, wm_ref, prev_ref, o_ref):
    del prev_ref
    _mm_body(gnf_ref, gmem_ref, wn_ref, wm_ref, o_ref)


def _make_mm(offset, bs):
    base = offset // TB
    row_specs = [
        pl.BlockSpec((TB, D_NODE), lambda i: (i, 0)),
        pl.BlockSpec((TB, D_MEM), lambda i: (i, 0)),
        pl.BlockSpec((D_NODE, D_EMB), lambda i: (0, 0)),
        pl.BlockSpec((D_MEM, D_EMB), lambda i: (0, 0)),
    ]
    if offset == 0:
        return pl.pallas_call(
            _mm_body,
            grid=(bs // TB,),
            in_specs=row_specs,
            out_specs=pl.BlockSpec((TB, D_EMB), lambda i: (i, 0)),
            out_shape=jax.ShapeDtypeStruct((B, D_EMB), jnp.float32),
        )
    return pl.pallas_call(
        _mm_body_aliased,
        grid=(bs // TB,),
        in_specs=row_specs + [pl.BlockSpec(memory_space=pl.ANY)],
        out_specs=pl.BlockSpec((TB, D_EMB), lambda i, b=base: (b + i, 0)),
        out_shape=jax.ShapeDtypeStruct((B, D_EMB), jnp.float32),
        input_output_aliases={4: 0},
    )


_mms = [_make_mm(OFFSETS[k], SPLITS[k]) for k in range(len(SPLITS))]


def kernel(memory, source_nodes, timestamps, node_features,
           W_node, W_mem, W_time, time_w, time_b):
    del timestamps, W_time, time_w, time_b
    gnf, gmem = _sc_gathers[0](node_features, memory, source_nodes)
    out = _mms[0](gnf, gmem, W_node, W_mem)
    for k in range(1, len(SPLITS)):
        gnf, gmem = _sc_gathers[k](node_features, memory, source_nodes)
        out = _mms[k](gnf, gmem, W_node, W_mem, out)
    return out


# speedup vs baseline: 1.0085x; 1.0085x over previous
import functools

import jax
import jax.numpy as jnp
from jax import lax
from jax.experimental import pallas as pl
from jax.experimental.pallas import tpu as pltpu
from jax.experimental.pallas import tpu_sc as plsc

B = 16384
D_NODE = 256
D_MEM = 512
D_EMB = 256

NC = 2
NS = 16
NW = NC * NS
CH = 32
SPLITS = (16384,)
NBUF = 4
DMA_FRAC_ROWS = 0
OFFSETS = tuple(sum(SPLITS[:k]) for k in range(len(SPLITS)))
assert sum(SPLITS) == B

_mesh = plsc.VectorSubcoreMesh(core_axis_name="c", subcore_axis_name="s")


def _make_sc_gather(offset, bs):
    bpw = bs // NW
    sr = bpw - DMA_FRAC_ROWS
    dr = DMA_FRAC_ROWS
    nchunk = sr // CH

    def body(nf_hbm, mem_hbm, idx_hbm, gnf_hbm, gmem_hbm,
             idx_v, idx_s, nf_buf, mem_buf, *sems):
        wid = lax.axis_index("s") * NC + lax.axis_index("c")
        base = wid * bpw
        pltpu.sync_copy(idx_hbm.at[pl.ds(offset + base, sr)], idx_v)
        sem_g = sems[:NBUF]
        sem_w = sems[NBUF:2 * NBUF]
        sem_d = sems[2 * NBUF]

        if dr:
            pltpu.sync_copy(idx_hbm.at[pl.ds(offset + base + sr, dr)], idx_s)

            def dma_row(i, carry):
                r = idx_s[i]
                pltpu.async_copy(nf_hbm.at[pl.ds(r, 1)],
                                 gnf_hbm.at[pl.ds(base + sr + i, 1)], sem_d)
                pltpu.async_copy(mem_hbm.at[pl.ds(r, 1)],
                                 gmem_hbm.at[pl.ds(base + sr + i, 1)], sem_d)
                return carry

            lax.fori_loop(0, dr, dma_row, 0, unroll=False)

        def fire_gather(c, p):
            ix = idx_v.at[pl.ds(c * CH, CH)]
            return (pltpu.async_copy(nf_hbm.at[ix], nf_buf.at[p], sem_g[p]),
                    pltpu.async_copy(mem_hbm.at[ix], mem_buf.at[p], sem_g[p]))

        def fire_write(c, p):
            o = base + c * CH
            return (pltpu.async_copy(nf_buf.at[p], gnf_hbm.at[pl.ds(o, CH)], sem_w[p]),
                    pltpu.async_copy(mem_buf.at[p], gmem_hbm.at[pl.ds(o, CH)], sem_w[p]))

        gather_cps = [None] * NBUF
        write_cps = [None] * NBUF
        for p in range(min(NBUF, nchunk)):
            gather_cps[p] = fire_gather(p, p)
        for c in range(nchunk):
            p = c % NBUF
            for cp in gather_cps[p]:
                cp.wait()
            write_cps[p] = fire_write(c, p)
            nxt = c + NBUF
            if nxt < nchunk:
                for cp in write_cps[p]:
                    cp.wait()
                gather_cps[p] = fire_gather(nxt, p)
        for p in range(NBUF):
            if write_cps[p] is not None:
                for cp in write_cps[p]:
                    cp.wait()
        if dr:
            pltpu.make_async_copy(
                nf_hbm.at[pl.ds(0, dr)],
                gnf_hbm.at[pl.ds(base + sr, dr)], sem_d).wait()
            pltpu.make_async_copy(
                mem_hbm.at[pl.ds(0, dr)],
                gmem_hbm.at[pl.ds(base + sr, dr)], sem_d).wait()

    return functools.partial(
        pl.kernel,
        out_type=(
            jax.ShapeDtypeStruct((bs, D_NODE), jnp.float32),
            jax.ShapeDtypeStruct((bs, D_MEM), jnp.float32),
        ),
        mesh=_mesh,
        scratch_types=[
            pltpu.VMEM((sr,), jnp.int32),
            pltpu.SMEM((max(dr, 1),), jnp.int32),
            pltpu.VMEM((NBUF, CH, D_NODE), jnp.float32),
            pltpu.VMEM((NBUF, CH, D_MEM), jnp.float32),
        ] + [pltpu.SemaphoreType.DMA] * (2 * NBUF + 1),
    )(body)


_sc_gathers = [_make_sc_gather(OFFSETS[k], SPLITS[k]) for k in range(len(SPLITS))]


TB = 4096


def _mm_body(gnf_ref, gmem_ref, wn_ref, wm_ref, o_ref):
    o_ref[...] = (
        jnp.dot(gnf_ref[...], wn_ref[...], preferred_element_type=jnp.float32)
        + jnp.dot(gmem_ref[...], wm_ref[...], preferred_element_type=jnp.float32)
    )


def _mm_body_aliased(gnf_ref, gmem_ref, wn_ref, wm_ref, prev_ref, o_ref):
    del prev_ref
    _mm_body(gnf_ref, gmem_ref, wn_ref, wm_ref, o_ref)


def _make_mm(offset, bs):
    base = offset // TB
    row_specs = [
        pl.BlockSpec((TB, D_NODE), lambda i: (i, 0)),
        pl.BlockSpec((TB, D_MEM), lambda i: (i, 0)),
        pl.BlockSpec((D_NODE, D_EMB), lambda i: (0, 0)),
        pl.BlockSpec((D_MEM, D_EMB), lambda i: (0, 0)),
    ]
    if offset == 0:
        return pl.pallas_call(
            _mm_body,
            grid=(bs // TB,),
            in_specs=row_specs,
            out_specs=pl.BlockSpec((TB, D_EMB), lambda i: (i, 0)),
            out_shape=jax.ShapeDtypeStruct((B, D_EMB), jnp.float32),
        )
    return pl.pallas_call(
        _mm_body_aliased,
        grid=(bs // TB,),
        in_specs=row_specs + [pl.BlockSpec(memory_space=pl.ANY)],
        out_specs=pl.BlockSpec((TB, D_EMB), lambda i, b=base: (b + i, 0)),
        out_shape=jax.ShapeDtypeStruct((B, D_EMB), jnp.float32),
        input_output_aliases={4: 0},
    )


_mms = [_make_mm(OFFSETS[k], SPLITS[k]) for k in range(len(SPLITS))]


def kernel(memory, source_nodes, timestamps, node_features,
           W_node, W_mem, W_time, time_w, time_b):
    del timestamps, W_time, time_w, time_b
    gnf, gmem = _sc_gathers[0](node_features, memory, source_nodes)
    out = _mms[0](gnf, gmem, W_node, W_mem)
    for k in range(1, len(SPLITS)):
        gnf, gmem = _sc_gathers[k](node_features, memory, source_nodes)
        out = _mms[k](gnf, gmem, W_node, W_mem, out)
    return out


# speedup vs baseline: 1.0181x; 1.0096x over previous
import functools

import jax
import jax.numpy as jnp
from jax import lax
from jax.experimental import pallas as pl
from jax.experimental.pallas import tpu as pltpu
from jax.experimental.pallas import tpu_sc as plsc

B = 16384
D_NODE = 256
D_MEM = 512
D_EMB = 256

NC = 2
NS = 16
NW = NC * NS
CH = 32
SPLITS = (16384,)
NBUF = 5
DMA_FRAC_ROWS = 0
OFFSETS = tuple(sum(SPLITS[:k]) for k in range(len(SPLITS)))
assert sum(SPLITS) == B

_mesh = plsc.VectorSubcoreMesh(core_axis_name="c", subcore_axis_name="s")


def _make_sc_gather(offset, bs):
    bpw = bs // NW
    sr = bpw - DMA_FRAC_ROWS
    dr = DMA_FRAC_ROWS
    nchunk = sr // CH

    def body(nf_hbm, mem_hbm, idx_hbm, gnf_hbm, gmem_hbm,
             idx_v, idx_s, nf_buf, mem_buf, *sems):
        wid = lax.axis_index("s") * NC + lax.axis_index("c")
        base = wid * bpw
        pltpu.sync_copy(idx_hbm.at[pl.ds(offset + base, sr)], idx_v)
        sem_g = sems[:NBUF]
        sem_w = sems[NBUF:2 * NBUF]
        sem_d = sems[2 * NBUF]

        if dr:
            pltpu.sync_copy(idx_hbm.at[pl.ds(offset + base + sr, dr)], idx_s)

            def dma_row(i, carry):
                r = idx_s[i]
                pltpu.async_copy(nf_hbm.at[pl.ds(r, 1)],
                                 gnf_hbm.at[pl.ds(base + sr + i, 1)], sem_d)
                pltpu.async_copy(mem_hbm.at[pl.ds(r, 1)],
                                 gmem_hbm.at[pl.ds(base + sr + i, 1)], sem_d)
                return carry

            lax.fori_loop(0, dr, dma_row, 0, unroll=False)

        def fire_gather(c, p):
            ix = idx_v.at[pl.ds(c * CH, CH)]
            return (pltpu.async_copy(nf_hbm.at[ix], nf_buf.at[p], sem_g[p]),
                    pltpu.async_copy(mem_hbm.at[ix], mem_buf.at[p], sem_g[p]))

        def fire_write(c, p):
            o = base + c * CH
            return (pltpu.async_copy(nf_buf.at[p], gnf_hbm.at[pl.ds(o, CH)], sem_w[p]),
                    pltpu.async_copy(mem_buf.at[p], gmem_hbm.at[pl.ds(o, CH)], sem_w[p]))

        gather_cps = [None] * NBUF
        write_cps = [None] * NBUF
        for p in range(min(NBUF, nchunk)):
            gather_cps[p] = fire_gather(p, p)
        for c in range(nchunk):
            p = c % NBUF
            for cp in gather_cps[p]:
                cp.wait()
            write_cps[p] = fire_write(c, p)
            nxt = c + NBUF
            if nxt < nchunk:
                for cp in write_cps[p]:
                    cp.wait()
                gather_cps[p] = fire_gather(nxt, p)
        for p in range(NBUF):
            if write_cps[p] is not None:
                for cp in write_cps[p]:
                    cp.wait()
        if dr:
            pltpu.make_async_copy(
                nf_hbm.at[pl.ds(0, dr)],
                gnf_hbm.at[pl.ds(base + sr, dr)], sem_d).wait()
            pltpu.make_async_copy(
                mem_hbm.at[pl.ds(0, dr)],
                gmem_hbm.at[pl.ds(base + sr, dr)], sem_d).wait()

    return functools.partial(
        pl.kernel,
        out_type=(
            jax.ShapeDtypeStruct((bs, D_NODE), jnp.float32),
            jax.ShapeDtypeStruct((bs, D_MEM), jnp.float32),
        ),
        mesh=_mesh,
        scratch_types=[
            pltpu.VMEM((sr,), jnp.int32),
            pltpu.SMEM((max(dr, 1),), jnp.int32),
            pltpu.VMEM((NBUF, CH, D_NODE), jnp.float32),
            pltpu.VMEM((NBUF, CH, D_MEM), jnp.float32),
        ] + [pltpu.SemaphoreType.DMA] * (2 * NBUF + 1),
    )(body)


_sc_gathers = [_make_sc_gather(OFFSETS[k], SPLITS[k]) for k in range(len(SPLITS))]


TB = 4096


def _mm_body(gnf_ref, gmem_ref, wn_ref, wm_ref, o_ref):
    o_ref[...] = (
        jnp.dot(gnf_ref[...], wn_ref[...], preferred_element_type=jnp.float32)
        + jnp.dot(gmem_ref[...], wm_ref[...], preferred_element_type=jnp.float32)
    )


def _mm_body_aliased(gnf_ref, gmem_ref, wn_ref, wm_ref, prev_ref, o_ref):
    del prev_ref
    _mm_body(gnf_ref, gmem_ref, wn_ref, wm_ref, o_ref)


def _make_mm(offset, bs):
    base = offset // TB
    row_specs = [
        pl.BlockSpec((TB, D_NODE), lambda i: (i, 0)),
        pl.BlockSpec((TB, D_MEM), lambda i: (i, 0)),
        pl.BlockSpec((D_NODE, D_EMB), lambda i: (0, 0)),
        pl.BlockSpec((D_MEM, D_EMB), lambda i: (0, 0)),
    ]
    if offset == 0:
        return pl.pallas_call(
            _mm_body,
            grid=(bs // TB,),
            in_specs=row_specs,
            out_specs=pl.BlockSpec((TB, D_EMB), lambda i: (i, 0)),
            out_shape=jax.ShapeDtypeStruct((B, D_EMB), jnp.float32),
        )
    return pl.pallas_call(
        _mm_body_aliased,
        grid=(bs // TB,),
        in_specs=row_specs + [pl.BlockSpec(memory_space=pl.ANY)],
        out_specs=pl.BlockSpec((TB, D_EMB), lambda i, b=base: (b + i, 0)),
        out_shape=jax.ShapeDtypeStruct((B, D_EMB), jnp.float32),
        input_output_aliases={4: 0},
    )


_mms = [_make_mm(OFFSETS[k], SPLITS[k]) for k in range(len(SPLITS))]


def kernel(memory, source_nodes, timestamps, node_features,
           W_node, W_mem, W_time, time_w, time_b):
    del timestamps, W_time, time_w, time_b
    gnf, gmem = _sc_gathers[0](node_features, memory, source_nodes)
    out = _mms[0](gnf, gmem, W_node, W_mem)
    for k in range(1, len(SPLITS)):
        gnf, gmem = _sc_gathers[k](node_features, memory, source_nodes)
        out = _mms[k](gnf, gmem, W_node, W_mem, out)
    return out
